# baseline (speedup 1.0000x reference)
import functools

import jax
import jax.numpy as jnp
from jax import lax
from jax.experimental import pallas as pl
from jax.experimental.pallas import tpu as pltpu
from jax.experimental.pallas import tpu_sc as plsc

NV = 40962
NV_PREV = 10242
NF = 81920
C = 256
OUT_CH = 128
BS = 2
LANES = 16
NGRP = C // LANES

NC, NSUB = 2, 16
NW = NC * NSUB

RPI = 16
NV_PAD = 41472
NF_PER_W = NF // NW
NV_PER_W = NV_PAD // NW

GSEC = 3 * NF
LSEC = NV_PAD * 7
FSEC = NV_PAD * 6
OFF_L, OFF_F = 0, LSEC

_MESH = plsc.VectorSubcoreMesh(
    core_axis_name="c", subcore_axis_name="s", num_cores=NC, num_subcores=NSUB)


def _wid():
  return lax.axis_index("s") * NC + lax.axis_index("c")


def _bcast(x):
  return jnp.broadcast_to(x, (LANES,))


def _wvec(ref, off):
  return ref[pl.ds(off, LANES)]


def _build_grad(supc=8):
  n_iters = NF_PER_W // RPI
  n_sup = n_iters // supc
  sec = supc * RPI * 3
  esec = supc * RPI

  @functools.partial(
      pl.kernel, mesh=_MESH,
      out_type=jax.ShapeDtypeStruct((NF, 2 * C), jnp.float32),
      scratch_types=(
          [pltpu.VMEM((3 * sec,), jnp.int32)] +
          [pltpu.VMEM((3 * sec + 6 * esec + LANES,), jnp.float32)] +
          [pltpu.VMEM((RPI * 9, C), jnp.float32) for _ in range(2)] +
          [pltpu.VMEM((RPI, C), jnp.float32) for _ in range(4)] +
          [pltpu.SemaphoreType.DMA for _ in range(7)]),
  )
  def grad_k(xt_hbm, gcols_hbm, wmeta_hbm, o_hbm,
             colbuf, wbuf, rows0, rows1, oew0, oew1, ons0, ons1,
             gsem0, gsem1, wsem_ew0, wsem_ew1, wsem_ns0, wsem_ns1, msem):
    rowsb = (rows0, rows1)
    oewb = (oew0, oew1)
    onsb = (ons0, ons1)
    gsems = (gsem0, gsem1)
    wsems = ((wsem_ew0, wsem_ns0), (wsem_ew1, wsem_ns1))
    base = _wid() * NF_PER_W

    def issue_gathers(i, p):
      return [pltpu.async_copy(
          xt_hbm.at[colbuf.at[pl.ds(d * sec + i * 48, 48)]],
          rowsb[p].at[pl.ds(d * 48, 48)], gsems[p]) for d in range(3)]

    def issue_meta(f0):
      for d in range(3):
        pltpu.async_copy(gcols_hbm.at[pl.ds(d * GSEC + f0 * 3, sec)],
                         colbuf.at[pl.ds(d * sec, sec)], msem)
        pltpu.async_copy(wmeta_hbm.at[pl.ds(d * GSEC + f0 * 3, sec)],
                         wbuf.at[pl.ds(d * sec, sec)], msem)
      for k in range(6):
        pltpu.async_copy(wmeta_hbm.at[pl.ds(3 * GSEC + k * NF + f0, esec)],
                         wbuf.at[pl.ds(3 * sec + k * esec, esec)], msem)

    def wait_meta(f0):
      for d in range(3):
        pltpu.make_async_copy(gcols_hbm.at[pl.ds(d * GSEC + f0 * 3, sec)],
                              colbuf.at[pl.ds(d * sec, sec)], msem).wait()
        pltpu.make_async_copy(wmeta_hbm.at[pl.ds(d * GSEC + f0 * 3, sec)],
                              wbuf.at[pl.ds(d * sec, sec)], msem).wait()
      for k in range(6):
        pltpu.make_async_copy(
            wmeta_hbm.at[pl.ds(3 * GSEC + k * NF + f0, esec)],
            wbuf.at[pl.ds(3 * sec + k * esec, esec)], msem).wait()

    def sup_body(s, carry):
      f0 = base + s * (supc * RPI)

      @pl.when(s == 0)
      def _prologue():
        issue_meta(f0)

      wait_meta(f0)
      gh = {0: issue_gathers(0, 0)}
      wh = {}
      for i in range(supc):
        p = i % 2
        if i + 1 < supc:
          gh[i + 1] = issue_gathers(i + 1, (i + 1) % 2)
        for h in gh.pop(i):
          h.wait()
        if i - 2 in wh:
          for h in wh.pop(i - 2):
            h.wait()

        def row_body(r, c2):
          off = i * (RPI * 3) + r * 3
          offe = i * RPI + r
          gvv = [_wvec(wbuf, d * sec + off) for d in range(3)]
          eww = [_wvec(wbuf, 3 * sec + d * esec + offe) for d in range(3)]
          nsw = [_wvec(wbuf, 3 * sec + (3 + d) * esec + offe) for d in range(3)]
          wew = [_bcast(gvv[d][k] * eww[d][0])
                 for d in range(3) for k in range(3)]
          wns = [_bcast(gvv[d][k] * nsw[d][0])
                 for d in range(3) for k in range(3)]
          for g in range(NGRP):
            acc_ew = None
            acc_ns = None
            for d in range(3):
              for k in range(3):
                rv = rowsb[p][d * 48 + r * 3 + k, pl.ds(g * LANES, LANES)]
                tew = wew[d * 3 + k] * rv
                tns = wns[d * 3 + k] * rv
                acc_ew = tew if acc_ew is None else acc_ew + tew
                acc_ns = tns if acc_ns is None else acc_ns + tns
            oewb[p][r, pl.ds(g * LANES, LANES)] = acc_ew
            onsb[p][r, pl.ds(g * LANES, LANES)] = acc_ns
          return c2

        lax.fori_loop(0, RPI, row_body, 0)
        ri = f0 + i * RPI
        wh[i] = [
            pltpu.async_copy(oewb[p], o_hbm.at[pl.ds(ri, RPI), pl.ds(0, C)],
                             wsems[p][0]),
            pltpu.async_copy(onsb[p], o_hbm.at[pl.ds(ri, RPI), pl.ds(C, C)],
                             wsems[p][1]),
        ]

      @pl.when(s + 1 < n_sup)
      def _prefetch():
        issue_meta(base + (s + 1) * (supc * RPI))

      for kk in sorted(wh):
        for h in wh.pop(kk):
          h.wait()
      return carry

    lax.fori_loop(0, n_sup, sup_body, 0)

  return grad_k


def _build_combine(nnz, off, n_outs, supc=9, tw=C, depth=2):
  n_iters = NV_PER_W // RPI
  n_sup = n_iters // supc
  ipi = RPI * nnz
  mlen = supc * ipi
  assert ipi <= 128 and ipi % 8 == 0 and tw == n_outs * C

  scratch = (
      [pltpu.VMEM((mlen,), jnp.int32)] +
      [pltpu.VMEM((mlen + LANES,), jnp.float32)] +
      [pltpu.VMEM((ipi, tw), jnp.float32) for _ in range(depth)] +
      [pltpu.VMEM((RPI, C), jnp.float32) for _ in range(2 * n_outs)] +
      [pltpu.SemaphoreType.DMA for _ in range(1 + depth + 2 * n_outs)])
  out_types = tuple(jax.ShapeDtypeStruct((NV_PAD, C), jnp.float32)
                    for _ in range(n_outs))

  @functools.partial(pl.kernel, mesh=_MESH,
                     out_type=out_types if n_outs > 1 else out_types[0],
                     scratch_types=scratch)
  def comb_k(*refs):
    tab = refs[0]
    mcols_hbm, mvals_hbm = refs[1:3]
    outs_hbm = refs[3:3 + n_outs]
    pos = 3 + n_outs
    colbuf, wbuf = refs[pos], refs[pos + 1]
    pos += 2
    rowsb = refs[pos:pos + depth]
    pos += depth
    outb = (refs[pos:pos + n_outs], refs[pos + n_outs:pos + 2 * n_outs])
    pos += 2 * n_outs
    gsems = refs[pos:pos + depth]
    pos += depth
    wsems = (refs[pos:pos + n_outs], refs[pos + n_outs:pos + 2 * n_outs])
    msem = refs[pos + 2 * n_outs]
    base = _wid() * NV_PER_W

    def issue_gathers(i, p):
      return [pltpu.async_copy(
          tab.at[colbuf.at[pl.ds(i * ipi, ipi)]], rowsb[p], gsems[p])]

    def issue_meta(r0):
      pltpu.async_copy(mcols_hbm.at[pl.ds(off + r0 * nnz, mlen)], colbuf,
                       msem)
      pltpu.async_copy(mvals_hbm.at[pl.ds(off + r0 * nnz, mlen)],
                       wbuf.at[pl.ds(0, mlen)], msem)

    def wait_meta(r0):
      pltpu.make_async_copy(mcols_hbm.at[pl.ds(off + r0 * nnz, mlen)],
                            colbuf, msem).wait()
      pltpu.make_async_copy(mvals_hbm.at[pl.ds(off + r0 * nnz, mlen)],
                            wbuf.at[pl.ds(0, mlen)], msem).wait()

    def sup_body(s, carry):
      r0 = base + s * (supc * RPI)

      @pl.when(s == 0)
      def _prologue():
        issue_meta(r0)

      wait_meta(r0)
      gh = {k: issue_gathers(k, k % depth) for k in range(depth - 1)}
      wh = {}
      for i in range(supc):
        p = i % depth
        po = i % 2
        nxt = i + depth - 1
        if nxt < supc:
          gh[nxt] = issue_gathers(nxt, nxt % depth)
        for h in gh.pop(i):
          h.wait()
        if i - 2 in wh:
          for h in wh.pop(i - 2):
            h.wait()

        def row_body(r, c2):
          wrow = _wvec(wbuf, i * ipi + r * nnz)
          wv = [_bcast(wrow[j]) for j in range(nnz)]
          for g in range(NGRP):
            accs = [None] * n_outs
            for j in range(nnz):
              for t in range(n_outs):
                rv = rowsb[p][r * nnz + j, pl.ds(t * C + g * LANES, LANES)]
                term = wv[j] * rv
                accs[t] = term if accs[t] is None else accs[t] + term
            for t in range(n_outs):
              outb[po][t][r, pl.ds(g * LANES, LANES)] = accs[t]
          return c2

        lax.fori_loop(0, RPI, row_body, 0)
        ri = r0 + i * RPI
        wh[i] = [pltpu.async_copy(outb[po][t], outs_hbm[t].at[pl.ds(ri, RPI)],
                                  wsems[po][t]) for t in range(n_outs)]

      @pl.when(s + 1 < n_sup)
      def _prefetch():
        issue_meta(base + (s + 1) * (supc * RPI))

      for kk in sorted(wh):
        for h in wh.pop(kk):
          h.wait()
      return carry

    lax.fori_loop(0, n_sup, sup_body, 0)

  return comb_k


_GRAD_K = _build_grad()
_LAP_K = _build_combine(7, off=OFF_L, n_outs=1, tw=C, depth=3)
_F2V_K = _build_combine(6, off=OFF_F, n_outs=2, tw=2 * C, depth=2)

_NB = 512
_NBLK = NV_PAD // _NB


def _xt_body(in_ref, o_ref):
  i = pl.program_id(0)
  t = in_ref[...].T
  rowv = lax.broadcasted_iota(jnp.int32, (_NB, C), 0) + i * _NB
  o_ref[...] = jnp.where(rowv < NV_PREV, t,
                         jnp.where(rowv < NV, 1.0, 0.0))


def _build_xt(input2d):
  n_in_blk = -(-NV_PREV // _NB) - 1
  return pl.pallas_call(
      _xt_body,
      grid=(_NBLK,),
      in_specs=[pl.BlockSpec((C, _NB),
                             lambda i: (0, jnp.minimum(i, n_in_blk)))],
      out_specs=pl.BlockSpec((_NB, C), lambda i: (i, 0)),
      out_shape=jax.ShapeDtypeStruct((NV_PAD, C), jnp.float32),
  )(input2d)


def _tc_body(x_ref, l_ref, e_ref, n_ref, c_ref, o_ref):
  feats = (x_ref, l_ref, e_ref, n_ref)
  acc = None
  for j in range(4):
    t = lax.dot_general(feats[j][...], c_ref[j], (((1,), (0,)), ((), ())),
                        preferred_element_type=jnp.float32)
    acc = t if acc is None else acc + t
  o_ref[...] = acc


def _tc_matmul(x_t, lap, gv_ew, gv_ns, cj):
  feat_spec = pl.BlockSpec((_NB, 128), lambda b, i: (i, b))
  return pl.pallas_call(
      _tc_body,
      grid=(BS, _NBLK),
      in_specs=[feat_spec, feat_spec, feat_spec, feat_spec,
                pl.BlockSpec((4, 128, OUT_CH), lambda b, i: (0, 0, 0))],
      out_specs=pl.BlockSpec((_NB, OUT_CH), lambda b, i: (i, b)),
      out_shape=jax.ShapeDtypeStruct((NV, BS * OUT_CH), jnp.float32),
  )(x_t, lap, gv_ew, gv_ns, cj)


def _pack_meta(L_cols, L_vals, F_cols, F_vals):
  zli = jnp.zeros((LSEC - NV * 7,), jnp.int32)
  zfi = jnp.zeros((FSEC - NV * 6,), jnp.int32)
  zlf = jnp.zeros((LSEC - NV * 7,), jnp.float32)
  zff = jnp.zeros((FSEC - NV * 6,), jnp.float32)
  return (jnp.concatenate([L_cols, zli, F_cols, zfi]),
          jnp.concatenate([L_vals, zlf, F_vals, zff]))


def _pack_gw(G_vals, EW, NS):
  return jnp.concatenate([G_vals, EW.T.reshape(-1), NS.T.reshape(-1)])


def kernel(input, coeffs, G_rows, G_cols, G_vals, L_rows, L_cols, L_vals,
           F_rows, F_cols, F_vals, NS, EW):
  bs, ch, _ = input.shape
  x_t = _build_xt(input.reshape(bs * ch, NV_PREV))
  mcols, mvals = _pack_meta(L_cols, L_vals, F_cols, F_vals)
  gw = _pack_gw(G_vals, EW, NS)

  gf = _GRAD_K(x_t, G_cols, gw)
  lap = _LAP_K(x_t, mcols, mvals)
  gv_ew, gv_ns = _F2V_K(gf, mcols, mvals)

  cj = coeffs.reshape(ch, 4, OUT_CH).transpose(1, 0, 2)
  out_v = _tc_matmul(x_t, lap, gv_ew, gv_ns, cj)
  return out_v.reshape(NV, bs, OUT_CH).transpose(1, 2, 0)
